# baseline (speedup 1.0000x reference)
import jax
import jax.numpy as jnp
from jax import lax
from jax.experimental import pallas as pl
from jax.experimental.pallas import tpu as pltpu
from jax.experimental.pallas import tpu_sc as plsc

_B = 4096
_L = 200
_H = 64
_LANES = 16
_NC = 2
_NS = 16
_NW = _NC * _NS
_BT = _B // 128
_ITEMS = _L * _BT
_IPW = _ITEMS // _NW
_NBUF = 3
_NT = 2
_EPS = 1e-12
_GRPS = 128 // _LANES


def _rsqrt_vec(x):
    i = plsc.bitcast(x, jnp.int32)
    i = jnp.int32(0x5F3759DF) - lax.shift_right_arithmetic(i, 1)
    y = plsc.bitcast(i, jnp.float32)
    for _ in range(2):
        y = y * (1.5 - 0.5 * x * y * y)
    return y


def _splat(ref, r, c):
    idx = jnp.full((_LANES,), r, jnp.int32)
    cdx = jnp.full((_LANES,), c, jnp.int32)
    return plsc.load_gather(ref, (idx, cdx))


def _sc_body(ids_hbm, table_hbm, pos_hbm, gamma_hbm, beta_hbm, out_hbm,
             pos_v, gb_v, idxr, bufs, tiles, isem, gsem, osem):
    w = lax.axis_index("s") * _NC + lax.axis_index("c")
    m0 = w * _IPW

    pltpu.sync_copy(pos_hbm, pos_v)
    pltpu.sync_copy(gamma_hbm, gb_v.at[0])
    pltpu.sync_copy(beta_hbm, gb_v.at[1])

    iota = lax.iota(jnp.int32, _LANES)
    rows_g = [iota + g * _LANES for g in range(_GRPS)]

    def lbt(m):
        return lax.div(m, _BT), lax.rem(m, _BT)

    def issue_idx(m, slot):
        l, bt = lbt(m)
        pltpu.async_copy(ids_hbm.at[l, pl.ds(bt * 128, 128)],
                         idxr.at[slot], isem.at[slot])

    def issue_gather(slot):
        pltpu.async_copy(table_hbm.at[idxr.at[slot]], bufs.at[slot],
                         gsem.at[slot])

    for j in range(_NBUF):
        issue_idx(m0 + j, j)
    for j in range(2):
        pltpu.make_async_copy(ids_hbm.at[0, pl.ds(0, 128)], idxr.at[j],
                              isem.at[j]).wait()
        issue_gather(j)

    def item_body(i, carry):
        b = lax.rem(i, _NBUF)
        t = lax.rem(i, _NT)
        m = m0 + i
        l, bt = lbt(m)

        pltpu.make_async_copy(table_hbm.at[idxr.at[b]], bufs.at[b],
                              gsem.at[b]).wait()

        @pl.when(i >= _NT)
        def _():
            plevel, pbt = lbt(m - _NT)
            pltpu.make_async_copy(tiles.at[t], out_hbm.at[plevel, :, pbt],
                                  osem.at[t]).wait()

        zero = jnp.zeros((_LANES,), jnp.float32)
        init = (tuple(zero for _ in range(_GRPS)),
                tuple(zero for _ in range(_GRPS)))

        def h_body(h, c):
            sums, sqs = c
            cdx = jnp.full((_LANES,), h, jnp.int32)
            p = _splat(pos_v, l, h)
            hh = lax.div(h, 8)
            hl = lax.rem(h, 8)
            ns, nq = [], []
            for g in range(_GRPS):
                x = plsc.load_gather(bufs.at[b], (rows_g[g], cdx)) + p
                tiles[t, hh, hl, pl.ds(g * _LANES, _LANES)] = x
                ns.append(sums[g] + x)
                nq.append(sqs[g] + x * x)
            return (tuple(ns), tuple(nq))

        sums, sqs = lax.fori_loop(0, _H, h_body, init)

        avs, cvs = [], []
        for g in range(_GRPS):
            mean = sums[g] * (1.0 / _H)
            var = jnp.maximum(sqs[g] * (1.0 / _H) - mean * mean, 0.0)
            a = _rsqrt_vec(var + _EPS)
            avs.append(a)
            cvs.append(mean * a)

        def h2_body(h, c):
            gam = _splat(gb_v, 0, h)
            bet = _splat(gb_v, 1, h)
            hh = lax.div(h, 8)
            hl = lax.rem(h, 8)
            for g in range(_GRPS):
                x = tiles[t, hh, hl, pl.ds(g * _LANES, _LANES)]
                y = (x * avs[g] - cvs[g]) * gam + bet
                tiles[t, hh, hl, pl.ds(g * _LANES, _LANES)] = y
            return c

        lax.fori_loop(0, _H, h2_body, 0)

        pltpu.async_copy(tiles.at[t], out_hbm.at[l, :, bt], osem.at[t])

        @pl.when(i + 2 < _IPW)
        def _():
            nb = lax.rem(i + 2, _NBUF)
            pltpu.make_async_copy(ids_hbm.at[0, pl.ds(0, 128)], idxr.at[nb],
                                  isem.at[nb]).wait()
            issue_gather(nb)

        @pl.when(i + 3 < _IPW)
        def _():
            issue_idx(m0 + i + 3, lax.rem(i + 3, _NBUF))

        return carry

    lax.fori_loop(0, _IPW, item_body, 0)

    for j in range(_NT):
        i = _IPW - _NT + j
        t = i % _NT
        l, bt = lbt(m0 + i)
        pltpu.make_async_copy(tiles.at[t], out_hbm.at[l, :, bt],
                              osem.at[t]).wait()


@jax.jit
def _sc_call(ids_t, item_table, pos_table, ln_gamma, ln_beta):
    mesh = plsc.VectorSubcoreMesh(
        core_axis_name="c", subcore_axis_name="s",
        num_cores=_NC, num_subcores=_NS)
    fn = pl.kernel(
        _sc_body,
        out_type=jax.ShapeDtypeStruct((_L, _H // 8, _BT, 8, 128),
                                      jnp.float32),
        mesh=mesh,
        compiler_params=pltpu.CompilerParams(
            needs_layout_passes=False, use_tc_tiling_on_sc=False),
        scratch_types=[
            pltpu.VMEM((_L, _H), jnp.float32),
            pltpu.VMEM((2, _H), jnp.float32),
            pltpu.VMEM((_NBUF, 128), jnp.int32),
            pltpu.VMEM((_NBUF, 128, _H), jnp.float32),
            pltpu.VMEM((_NT, 8, 8, 128), jnp.float32),
            pltpu.SemaphoreType.DMA((_NBUF,)),
            pltpu.SemaphoreType.DMA((_NBUF,)),
            pltpu.SemaphoreType.DMA((_NT,)),
        ],
    )
    return fn(ids_t, item_table, pos_table, ln_gamma, ln_beta)


def kernel(input_ids, item_table, pos_table, ln_gamma, ln_beta):
    out5 = _sc_call(input_ids.T, item_table, pos_table, ln_gamma, ln_beta)
    return out5.transpose(2, 4, 0, 1, 3).reshape(_B, _L, _H)


# speedup vs baseline: 1.4352x; 1.4352x over previous
import jax
import jax.numpy as jnp
from jax import lax
from jax.experimental import pallas as pl
from jax.experimental.pallas import tpu as pltpu
from jax.experimental.pallas import tpu_sc as plsc

_B = 4096
_L = 200
_H = 64
_LANES = 16
_NC = 2
_NS = 16
_NW = _NC * _NS
_BT = _B // 128
_ITEMS = _L * _BT
_IPW = _ITEMS // _NW
_NBUF = 3
_NT = 2
_EPS = 1e-12
_GRPS = 128 // _LANES


def _rsqrt_vec(x):
    i = plsc.bitcast(x, jnp.int32)
    i = jnp.int32(0x5F3759DF) - lax.shift_right_arithmetic(i, 1)
    y = plsc.bitcast(i, jnp.float32)
    for _ in range(2):
        y = y * (1.5 - 0.5 * x * y * y)
    return y


def _splat(ref, r, c):
    idx = jnp.full((_LANES,), r, jnp.int32)
    cdx = jnp.full((_LANES,), c, jnp.int32)
    return plsc.load_gather(ref, (idx, cdx))


def _sc_body(ids_hbm, table_hbm, pos_hbm, gamma_hbm, beta_hbm, out_hbm,
             pos_v, gb_v, idxr, bufs, tiles, isem, gsem, osem):
    w = lax.axis_index("s") * _NC + lax.axis_index("c")
    m0 = w * _IPW

    pltpu.sync_copy(pos_hbm, pos_v)
    pltpu.sync_copy(gamma_hbm, gb_v.at[0])
    pltpu.sync_copy(beta_hbm, gb_v.at[1])

    iota = lax.iota(jnp.int32, _LANES)
    rows_g = [iota + g * _LANES for g in range(_GRPS)]

    def lbt(m):
        return lax.div(m, _BT), lax.rem(m, _BT)

    def issue_idx(m, slot):
        l, bt = lbt(m)
        pltpu.async_copy(ids_hbm.at[l, pl.ds(bt * 128, 128)],
                         idxr.at[slot], isem.at[slot])

    def issue_gather(slot):
        pltpu.async_copy(table_hbm.at[idxr.at[slot]], bufs.at[slot],
                         gsem.at[slot])

    for j in range(_NBUF):
        issue_idx(m0 + j, j)
    for j in range(2):
        pltpu.make_async_copy(ids_hbm.at[0, pl.ds(0, 128)], idxr.at[j],
                              isem.at[j]).wait()
        issue_gather(j)

    def item_body(i, carry):
        b = lax.rem(i, _NBUF)
        t = lax.rem(i, _NT)
        m = m0 + i
        l, bt = lbt(m)

        pltpu.make_async_copy(table_hbm.at[idxr.at[b]], bufs.at[b],
                              gsem.at[b]).wait()

        @pl.when(i >= _NT)
        def _():
            plevel, pbt = lbt(m - _NT)
            pltpu.make_async_copy(tiles.at[t], out_hbm.at[plevel, :, pbt],
                                  osem.at[t]).wait()

        zero = jnp.zeros((_LANES,), jnp.float32)
        init = (tuple(zero for _ in range(_GRPS)),
                tuple(zero for _ in range(_GRPS)))

        @plsc.parallel_loop(0, _H, unroll=4, carry=init)
        def pass1(h, c):
            sums, sqs = c
            cdx = jnp.full((_LANES,), h, jnp.int32)
            p = _splat(pos_v, l, h)
            hh = lax.div(h, 8)
            hl = lax.rem(h, 8)
            ns, nq = [], []
            for g in range(_GRPS):
                x = plsc.load_gather(bufs.at[b], (rows_g[g], cdx)) + p
                tiles[t, hh, hl, pl.ds(g * _LANES, _LANES)] = x
                ns.append(sums[g] + x)
                nq.append(sqs[g] + x * x)
            return (tuple(ns), tuple(nq))

        sums, sqs = pass1

        avs, cvs = [], []
        for g in range(_GRPS):
            mean = sums[g] * (1.0 / _H)
            var = jnp.maximum(sqs[g] * (1.0 / _H) - mean * mean, 0.0)
            a = _rsqrt_vec(var + _EPS)
            avs.append(a)
            cvs.append(mean * a)

        @plsc.parallel_loop(0, _H, unroll=4)
        def pass2(h):
            gam = _splat(gb_v, 0, h)
            bet = _splat(gb_v, 1, h)
            hh = lax.div(h, 8)
            hl = lax.rem(h, 8)
            for g in range(_GRPS):
                x = tiles[t, hh, hl, pl.ds(g * _LANES, _LANES)]
                y = (x * avs[g] - cvs[g]) * gam + bet
                tiles[t, hh, hl, pl.ds(g * _LANES, _LANES)] = y

        del pass2

        pltpu.async_copy(tiles.at[t], out_hbm.at[l, :, bt], osem.at[t])

        @pl.when(i + 2 < _IPW)
        def _():
            nb = lax.rem(i + 2, _NBUF)
            pltpu.make_async_copy(ids_hbm.at[0, pl.ds(0, 128)], idxr.at[nb],
                                  isem.at[nb]).wait()
            issue_gather(nb)

        @pl.when(i + 3 < _IPW)
        def _():
            issue_idx(m0 + i + 3, lax.rem(i + 3, _NBUF))

        return carry

    lax.fori_loop(0, _IPW, item_body, 0)

    for j in range(_NT):
        i = _IPW - _NT + j
        t = i % _NT
        l, bt = lbt(m0 + i)
        pltpu.make_async_copy(tiles.at[t], out_hbm.at[l, :, bt],
                              osem.at[t]).wait()


@jax.jit
def _sc_call(ids_t, item_table, pos_table, ln_gamma, ln_beta):
    mesh = plsc.VectorSubcoreMesh(
        core_axis_name="c", subcore_axis_name="s",
        num_cores=_NC, num_subcores=_NS)
    fn = pl.kernel(
        _sc_body,
        out_type=jax.ShapeDtypeStruct((_L, _H // 8, _BT, 8, 128),
                                      jnp.float32),
        mesh=mesh,
        compiler_params=pltpu.CompilerParams(
            needs_layout_passes=False, use_tc_tiling_on_sc=False),
        scratch_types=[
            pltpu.VMEM((_L, _H), jnp.float32),
            pltpu.VMEM((2, _H), jnp.float32),
            pltpu.VMEM((_NBUF, 128), jnp.int32),
            pltpu.VMEM((_NBUF, 128, _H), jnp.float32),
            pltpu.VMEM((_NT, 8, 8, 128), jnp.float32),
            pltpu.SemaphoreType.DMA((_NBUF,)),
            pltpu.SemaphoreType.DMA((_NBUF,)),
            pltpu.SemaphoreType.DMA((_NT,)),
        ],
    )
    return fn(ids_t, item_table, pos_table, ln_gamma, ln_beta)


def kernel(input_ids, item_table, pos_table, ln_gamma, ln_beta):
    out5 = _sc_call(input_ids.T, item_table, pos_table, ln_gamma, ln_beta)
    return out5.transpose(2, 4, 0, 1, 3).reshape(_B, _L, _H)


# speedup vs baseline: 1.9890x; 1.3859x over previous
import jax
import jax.numpy as jnp
from jax import lax
from jax.experimental import pallas as pl
from jax.experimental.pallas import tpu as pltpu
from jax.experimental.pallas import tpu_sc as plsc

_B = 4096
_L = 200
_H = 64
_LANES = 16
_NC = 2
_NS = 16
_NW = _NC * _NS
_BT = _B // 128
_ITEMS = _L * _BT
_IPW = _ITEMS // _NW
_NBUF = 4
_NT = 2
_EPS = 1e-12
_GRPS = 128 // _LANES


def _rsqrt_vec(x):
    i = plsc.bitcast(x, jnp.int32)
    i = jnp.int32(0x5F3759DF) - lax.shift_right_arithmetic(i, 1)
    y = plsc.bitcast(i, jnp.float32)
    for _ in range(2):
        y = y * (1.5 - 0.5 * x * y * y)
    return y


def _splat(ref, r, c):
    idx = jnp.full((_LANES,), r, jnp.int32)
    cdx = jnp.full((_LANES,), c, jnp.int32)
    return plsc.load_gather(ref, (idx, cdx))


def _sc_body(ids_hbm, table_hbm, pos_hbm, gamma_hbm, beta_hbm, out_hbm,
             pos_v, gb_v, idxr, bufs, tiles, isem, gsem, osem):
    w = lax.axis_index("s") * _NC + lax.axis_index("c")
    m0 = w * _IPW

    pltpu.sync_copy(pos_hbm, pos_v)
    pltpu.sync_copy(gamma_hbm, gb_v.at[0])
    pltpu.sync_copy(beta_hbm, gb_v.at[1])

    iota = lax.iota(jnp.int32, _LANES)
    rows_g = [iota + g * _LANES for g in range(_GRPS)]

    def lbt(m):
        return lax.div(m, _BT), lax.rem(m, _BT)

    def issue_idx(m, slot):
        l, bt = lbt(m)
        pltpu.async_copy(ids_hbm.at[l, pl.ds(bt * 128, 128)],
                         idxr.at[slot], isem.at[slot])

    def issue_gather(slot):
        pltpu.async_copy(table_hbm.at[idxr.at[slot]], bufs.at[slot],
                         gsem.at[slot])

    for j in range(_NBUF):
        issue_idx(m0 + j, j)
    for j in range(3):
        pltpu.make_async_copy(ids_hbm.at[0, pl.ds(0, 128)], idxr.at[j],
                              isem.at[j]).wait()
        issue_gather(j)

    def item_body(i, carry):
        b = lax.rem(i, _NBUF)
        t = lax.rem(i, _NT)
        m = m0 + i
        l, bt = lbt(m)

        pltpu.make_async_copy(table_hbm.at[idxr.at[b]], bufs.at[b],
                              gsem.at[b]).wait()

        @pl.when(i >= _NT)
        def _():
            plevel, pbt = lbt(m - _NT)
            pltpu.make_async_copy(tiles.at[t], out_hbm.at[plevel, :, pbt],
                                  osem.at[t]).wait()

        zero = jnp.zeros((_LANES,), jnp.float32)
        init = (tuple(zero for _ in range(_GRPS)),
                tuple(zero for _ in range(_GRPS)))

        lsplat = jnp.full((_LANES,), l, jnp.int32)

        @plsc.parallel_loop(0, _H, unroll=4, carry=init)
        def pass1(h, c):
            sums, sqs = c
            hd = h + iota
            hd = jnp.where(hd >= _H, hd - _H, hd)
            p = plsc.load_gather(pos_v, (lsplat, hd))
            hd_hi = lax.shift_right_logical(hd, 3)
            hd_lo = lax.bitwise_and(hd, 7)
            ns, nq = [], []
            for g in range(_GRPS):
                x = plsc.load_gather(bufs.at[b], (rows_g[g], hd)) + p
                plsc.store_scatter(tiles.at[t],
                                   (hd_hi, hd_lo, rows_g[g]), x)
                ns.append(sums[g] + x)
                nq.append(sqs[g] + x * x)
            return (tuple(ns), tuple(nq))

        sums, sqs = pass1

        avs, cvs = [], []
        for g in range(_GRPS):
            mean = sums[g] * (1.0 / _H)
            var = jnp.maximum(sqs[g] * (1.0 / _H) - mean * mean, 0.0)
            a = _rsqrt_vec(var + _EPS)
            avs.append(a)
            cvs.append(mean * a)

        @plsc.parallel_loop(0, _H, unroll=4)
        def pass2(h):
            gam = _splat(gb_v, 0, h)
            bet = _splat(gb_v, 1, h)
            hh = lax.div(h, 8)
            hl = lax.rem(h, 8)
            for g in range(_GRPS):
                x = tiles[t, hh, hl, pl.ds(g * _LANES, _LANES)]
                y = (x * avs[g] - cvs[g]) * gam + bet
                tiles[t, hh, hl, pl.ds(g * _LANES, _LANES)] = y

        del pass2

        pltpu.async_copy(tiles.at[t], out_hbm.at[l, :, bt], osem.at[t])

        @pl.when(i + 3 < _IPW)
        def _():
            nb = lax.rem(i + 3, _NBUF)
            pltpu.make_async_copy(ids_hbm.at[0, pl.ds(0, 128)], idxr.at[nb],
                                  isem.at[nb]).wait()
            issue_gather(nb)

        @pl.when(i + 4 < _IPW)
        def _():
            issue_idx(m0 + i + 4, lax.rem(i + 4, _NBUF))

        return carry

    lax.fori_loop(0, _IPW, item_body, 0)

    for j in range(_NT):
        i = _IPW - _NT + j
        t = i % _NT
        l, bt = lbt(m0 + i)
        pltpu.make_async_copy(tiles.at[t], out_hbm.at[l, :, bt],
                              osem.at[t]).wait()


@jax.jit
def _sc_call(ids_t, item_table, pos_table, ln_gamma, ln_beta):
    mesh = plsc.VectorSubcoreMesh(
        core_axis_name="c", subcore_axis_name="s",
        num_cores=_NC, num_subcores=_NS)
    fn = pl.kernel(
        _sc_body,
        out_type=jax.ShapeDtypeStruct((_L, _H // 8, _BT, 8, 128),
                                      jnp.float32),
        mesh=mesh,
        compiler_params=pltpu.CompilerParams(
            needs_layout_passes=False, use_tc_tiling_on_sc=False),
        scratch_types=[
            pltpu.VMEM((_L, _H), jnp.float32),
            pltpu.VMEM((2, _H), jnp.float32),
            pltpu.VMEM((_NBUF, 128), jnp.int32),
            pltpu.VMEM((_NBUF, 128, _H), jnp.float32),
            pltpu.VMEM((_NT, 8, 8, 128), jnp.float32),
            pltpu.SemaphoreType.DMA((_NBUF,)),
            pltpu.SemaphoreType.DMA((_NBUF,)),
            pltpu.SemaphoreType.DMA((_NT,)),
        ],
    )
    return fn(ids_t, item_table, pos_table, ln_gamma, ln_beta)


def kernel(input_ids, item_table, pos_table, ln_gamma, ln_beta):
    out5 = _sc_call(input_ids.T, item_table, pos_table, ln_gamma, ln_beta)
    return out5.transpose(2, 4, 0, 1, 3).reshape(_B, _L, _H)


# speedup vs baseline: 2.3327x; 1.1728x over previous
import jax
import jax.numpy as jnp
from jax import lax
from jax.experimental import pallas as pl
from jax.experimental.pallas import tpu as pltpu
from jax.experimental.pallas import tpu_sc as plsc

_B = 4096
_L = 200
_H = 64
_LANES = 16
_NC = 2
_NS = 16
_NW = _NC * _NS
_BT = _B // 128
_ITEMS = _L * _BT
_IPW = _ITEMS // _NW
_NBUF = 4
_NT = 2
_EPS = 1e-12
_GRPS = 128 // _LANES


def _rsqrt_vec(x):
    i = plsc.bitcast(x, jnp.int32)
    i = jnp.int32(0x5F3759DF) - lax.shift_right_arithmetic(i, 1)
    y = plsc.bitcast(i, jnp.float32)
    for _ in range(2):
        y = y * (1.5 - 0.5 * x * y * y)
    return y


def _splat(ref, r, c):
    idx = jnp.full((_LANES,), r, jnp.int32)
    cdx = jnp.full((_LANES,), c, jnp.int32)
    return plsc.load_gather(ref, (idx, cdx))


def _sc_body(ids_hbm, table_hbm, pos_hbm, gamma_hbm, beta_hbm, out_hbm,
             pos_v, gb_v, idxr, bufs, tiles, isem, gsem, osem):
    w = lax.axis_index("s") * _NC + lax.axis_index("c")
    m0 = w * _IPW

    pltpu.sync_copy(pos_hbm, pos_v)
    pltpu.sync_copy(gamma_hbm, gb_v.at[0])
    pltpu.sync_copy(beta_hbm, gb_v.at[1])

    iota = lax.iota(jnp.int32, _LANES)
    rows_g = [iota + g * _LANES for g in range(_GRPS)]

    def lbt(m):
        return lax.div(m, _BT), lax.rem(m, _BT)

    def issue_idx(m, slot):
        l, bt = lbt(m)
        pltpu.async_copy(ids_hbm.at[l, pl.ds(bt * 128, 128)],
                         idxr.at[slot], isem.at[slot])

    def issue_gather(slot):
        pltpu.async_copy(table_hbm.at[idxr.at[slot]], bufs.at[slot],
                         gsem.at[slot])

    for j in range(_NBUF):
        issue_idx(m0 + j, j)
    for j in range(3):
        pltpu.make_async_copy(ids_hbm.at[0, pl.ds(0, 128)], idxr.at[j],
                              isem.at[j]).wait()
        issue_gather(j)

    def item_body(i, carry):
        b = lax.rem(i, _NBUF)
        t = lax.rem(i, _NT)
        m = m0 + i
        l, bt = lbt(m)

        pltpu.make_async_copy(table_hbm.at[idxr.at[b]], bufs.at[b],
                              gsem.at[b]).wait()

        @pl.when(i >= _NT)
        def _():
            plevel, pbt = lbt(m - _NT)
            for ht in range(8):
                pltpu.make_async_copy(tiles.at[t, pl.ds(ht * 8, 8)],
                                      out_hbm.at[plevel, ht, pbt],
                                      osem.at[t]).wait()

        zero = jnp.zeros((_LANES,), jnp.float32)

        lsplat = jnp.full((_LANES,), l, jnp.int32)

        def half_pass1(g0):
            init = (tuple(zero for _ in range(4)),
                    tuple(zero for _ in range(4)))

            @plsc.parallel_loop(0, _H, unroll=4, carry=init)
            def hp(h, c):
                sums, sqs = c
                hd = h + iota
                hd = jnp.where(hd >= _H, hd - _H, hd)
                p = plsc.load_gather(pos_v, (lsplat, hd))
                ns, nq = [], []
                for k in range(4):
                    g = g0 + k
                    x = plsc.load_gather(bufs.at[b], (rows_g[g], hd)) + p
                    plsc.store_scatter(tiles.at[t], (hd, rows_g[g]), x)
                    ns.append(sums[k] + x)
                    nq.append(sqs[k] + x * x)
                return (tuple(ns), tuple(nq))

            return hp

        s_lo, q_lo = half_pass1(0)
        s_hi, q_hi = half_pass1(4)
        sums = tuple(s_lo) + tuple(s_hi)
        sqs = tuple(q_lo) + tuple(q_hi)

        avs, cvs = [], []
        for g in range(_GRPS):
            mean = sums[g] * (1.0 / _H)
            var = jnp.maximum(sqs[g] * (1.0 / _H) - mean * mean, 0.0)
            a = _rsqrt_vec(var + _EPS)
            avs.append(a)
            cvs.append(mean * a)

        @plsc.parallel_loop(0, _H, unroll=4)
        def pass2(h):
            gam = _splat(gb_v, 0, h)
            bet = _splat(gb_v, 1, h)
            for g in range(_GRPS):
                x = tiles[t, h, pl.ds(g * _LANES, _LANES)]
                y = (x * avs[g] - cvs[g]) * gam + bet
                tiles[t, h, pl.ds(g * _LANES, _LANES)] = y

        del pass2

        for ht in range(8):
            pltpu.async_copy(tiles.at[t, pl.ds(ht * 8, 8)],
                             out_hbm.at[l, ht, bt], osem.at[t])

        @pl.when(i + 3 < _IPW)
        def _():
            nb = lax.rem(i + 3, _NBUF)
            pltpu.make_async_copy(ids_hbm.at[0, pl.ds(0, 128)], idxr.at[nb],
                                  isem.at[nb]).wait()
            issue_gather(nb)

        @pl.when(i + 4 < _IPW)
        def _():
            issue_idx(m0 + i + 4, lax.rem(i + 4, _NBUF))

        return carry

    lax.fori_loop(0, _IPW, item_body, 0)

    for j in range(_NT):
        i = _IPW - _NT + j
        t = i % _NT
        l, bt = lbt(m0 + i)
        for ht in range(8):
            pltpu.make_async_copy(tiles.at[t, pl.ds(ht * 8, 8)],
                                  out_hbm.at[l, ht, bt], osem.at[t]).wait()


@jax.jit
def _sc_call(ids_t, item_table, pos_table, ln_gamma, ln_beta):
    mesh = plsc.VectorSubcoreMesh(
        core_axis_name="c", subcore_axis_name="s",
        num_cores=_NC, num_subcores=_NS)
    fn = pl.kernel(
        _sc_body,
        out_type=jax.ShapeDtypeStruct((_L, _H // 8, _BT, 8, 128),
                                      jnp.float32),
        mesh=mesh,
        compiler_params=pltpu.CompilerParams(
            needs_layout_passes=False, use_tc_tiling_on_sc=False),
        scratch_types=[
            pltpu.VMEM((_L, _H), jnp.float32),
            pltpu.VMEM((2, _H), jnp.float32),
            pltpu.VMEM((_NBUF, 128), jnp.int32),
            pltpu.VMEM((_NBUF, 128, _H), jnp.float32),
            pltpu.VMEM((_NT, _H, 128), jnp.float32),
            pltpu.SemaphoreType.DMA((_NBUF,)),
            pltpu.SemaphoreType.DMA((_NBUF,)),
            pltpu.SemaphoreType.DMA((_NT,)),
        ],
    )
    return fn(ids_t, item_table, pos_table, ln_gamma, ln_beta)


def kernel(input_ids, item_table, pos_table, ln_gamma, ln_beta):
    out5 = _sc_call(input_ids.T, item_table, pos_table, ln_gamma, ln_beta)
    return out5.transpose(2, 4, 0, 1, 3).reshape(_B, _L, _H)


# speedup vs baseline: 2.3461x; 1.0057x over previous
import jax
import jax.numpy as jnp
from jax import lax
from jax.experimental import pallas as pl
from jax.experimental.pallas import tpu as pltpu
from jax.experimental.pallas import tpu_sc as plsc

_B = 4096
_L = 200
_H = 64
_LANES = 16
_NC = 2
_NS = 16
_NW = _NC * _NS
_BT = _B // 128
_ITEMS = _L * _BT
_IPW = _ITEMS // _NW
_NBUF = 4
_NT = 2
_EPS = 1e-12
_GRPS = 128 // _LANES


def _rsqrt_vec(x):
    i = plsc.bitcast(x, jnp.int32)
    i = jnp.int32(0x5F3759DF) - lax.shift_right_arithmetic(i, 1)
    y = plsc.bitcast(i, jnp.float32)
    y = y * (1.5 - 0.5 * x * y * y)
    return y * (1.5 - 0.5 * x * y * y)


def _splat(ref, r, c):
    idx = jnp.full((_LANES,), r, jnp.int32)
    cdx = jnp.full((_LANES,), c, jnp.int32)
    return plsc.load_gather(ref, (idx, cdx))


def _sc_body(ids_hbm, table_hbm, pos_hbm, gamma_hbm, beta_hbm, out_hbm,
             pos_v, gb_v, idxr, bufs, tiles, isem, gsem, osem):
    w = lax.axis_index("s") * _NC + lax.axis_index("c")
    m0 = w * _IPW

    pltpu.sync_copy(pos_hbm, pos_v)
    pltpu.sync_copy(gamma_hbm, gb_v.at[0])
    pltpu.sync_copy(beta_hbm, gb_v.at[1])

    iota = lax.iota(jnp.int32, _LANES)
    rows_g = [iota + g * _LANES for g in range(_GRPS)]

    def lbt(m):
        return lax.div(m, _BT), lax.rem(m, _BT)

    def issue_idx(m, slot):
        l, bt = lbt(m)
        pltpu.async_copy(ids_hbm.at[l, pl.ds(bt * 128, 128)],
                         idxr.at[slot], isem.at[slot])

    def issue_gather(slot):
        pltpu.async_copy(table_hbm.at[idxr.at[slot]], bufs.at[slot],
                         gsem.at[slot])

    for j in range(_NBUF):
        issue_idx(m0 + j, j)
    for j in range(3):
        pltpu.make_async_copy(ids_hbm.at[0, pl.ds(0, 128)], idxr.at[j],
                              isem.at[j]).wait()
        issue_gather(j)

    def item_body(i, carry):
        b = lax.rem(i, _NBUF)
        t = lax.rem(i, _NT)
        m = m0 + i
        l, bt = lbt(m)

        pltpu.make_async_copy(table_hbm.at[idxr.at[b]], bufs.at[b],
                              gsem.at[b]).wait()

        @pl.when(i >= _NT)
        def _():
            pltpu.make_async_copy(table_hbm.at[pl.ds(0, 128)], bufs.at[b],
                                  osem.at[t]).wait()

        zero = jnp.zeros((_LANES,), jnp.float32)

        lsplat = jnp.full((_LANES,), l, jnp.int32)

        def half_pass1(g0):
            init = (tuple(zero for _ in range(4)),
                    tuple(zero for _ in range(4)))

            @plsc.parallel_loop(0, _H, unroll=4, carry=init)
            def hp(h, c):
                sums, sqs = c
                hd = h + iota
                hd = jnp.where(hd >= _H, hd - _H, hd)
                p = plsc.load_gather(pos_v, (lsplat, hd))
                ns, nq = [], []
                for k in range(4):
                    g = g0 + k
                    x = plsc.load_gather(bufs.at[b], (rows_g[g], hd)) + p
                    plsc.store_scatter(tiles.at[t], (hd, rows_g[g]), x)
                    ns.append(sums[k] + x)
                    nq.append(sqs[k] + x * x)
                return (tuple(ns), tuple(nq))

            return hp

        s_lo, q_lo = half_pass1(0)
        s_hi, q_hi = half_pass1(4)
        sums = tuple(s_lo) + tuple(s_hi)
        sqs = tuple(q_lo) + tuple(q_hi)

        avs, cvs = [], []
        for g in range(_GRPS):
            mean = sums[g] * (1.0 / _H)
            var = jnp.maximum(sqs[g] * (1.0 / _H) - mean * mean, 0.0)
            a = _rsqrt_vec(var + _EPS)
            avs.append(a)
            cvs.append(mean * a)

        @plsc.parallel_loop(0, _H, unroll=4)
        def pass2(h):
            gam = _splat(gb_v, 0, h)
            bet = _splat(gb_v, 1, h)
            for g in range(_GRPS):
                x = tiles[t, h, pl.ds(g * _LANES, _LANES)]
                y = (x * avs[g] - cvs[g]) * gam + bet
                tiles[t, h, pl.ds(g * _LANES, _LANES)] = y

        del pass2

        for ht in range(8):
            pltpu.async_copy(tiles.at[t, pl.ds(ht * 8, 8)],
                             out_hbm.at[l, ht, bt], osem.at[t])

        @pl.when(i + 3 < _IPW)
        def _():
            nb = lax.rem(i + 3, _NBUF)
            pltpu.make_async_copy(ids_hbm.at[0, pl.ds(0, 128)], idxr.at[nb],
                                  isem.at[nb]).wait()
            issue_gather(nb)

        @pl.when(i + 4 < _IPW)
        def _():
            issue_idx(m0 + i + 4, lax.rem(i + 4, _NBUF))

        return carry

    lax.fori_loop(0, _IPW, item_body, 0)

    for j in range(_NT):
        t = (_IPW - _NT + j) % _NT
        pltpu.make_async_copy(table_hbm.at[pl.ds(0, 128)], bufs.at[0],
                              osem.at[t]).wait()


@jax.jit
def _sc_call(ids_t, item_table, pos_table, ln_gamma, ln_beta):
    mesh = plsc.VectorSubcoreMesh(
        core_axis_name="c", subcore_axis_name="s",
        num_cores=_NC, num_subcores=_NS)
    fn = pl.kernel(
        _sc_body,
        out_type=jax.ShapeDtypeStruct((_L, _H // 8, _BT, 8, 128),
                                      jnp.float32),
        mesh=mesh,
        compiler_params=pltpu.CompilerParams(
            needs_layout_passes=False, use_tc_tiling_on_sc=False),
        scratch_types=[
            pltpu.VMEM((_L, _H), jnp.float32),
            pltpu.VMEM((2, _H), jnp.float32),
            pltpu.VMEM((_NBUF, 128), jnp.int32),
            pltpu.VMEM((_NBUF, 128, _H), jnp.float32),
            pltpu.VMEM((_NT, _H, 128), jnp.float32),
            pltpu.SemaphoreType.DMA((_NBUF,)),
            pltpu.SemaphoreType.DMA((_NBUF,)),
            pltpu.SemaphoreType.DMA((_NT,)),
        ],
    )
    return fn(ids_t, item_table, pos_table, ln_gamma, ln_beta)


def kernel(input_ids, item_table, pos_table, ln_gamma, ln_beta):
    out5 = _sc_call(input_ids.T, item_table, pos_table, ln_gamma, ln_beta)
    return out5.transpose(2, 4, 0, 1, 3).reshape(_B, _L, _H)


# speedup vs baseline: 3.9743x; 1.6940x over previous
import jax
import jax.numpy as jnp
from jax import lax
from jax.experimental import pallas as pl
from jax.experimental.pallas import tpu as pltpu
from jax.experimental.pallas import tpu_sc as plsc

_B = 4096
_L = 200
_H = 64
_LANES = 16
_NC = 2
_NS = 16
_NW = _NC * _NS
_BT = _B // 128
_ITEMS = _L * _BT
_IPW = _ITEMS // _NW
_NBUF = 4
_NT = 2
_EPS = 1e-12
_GRPS = 128 // _LANES


def _rsqrt_vec(x):
    i = plsc.bitcast(x, jnp.int32)
    i = jnp.int32(0x5F3759DF) - lax.shift_right_arithmetic(i, 1)
    y = plsc.bitcast(i, jnp.float32)
    y = y * (1.5 - 0.5 * x * y * y)
    return y * (1.5 - 0.5 * x * y * y)


def _splat(ref, r, c):
    idx = jnp.full((_LANES,), r, jnp.int32)
    cdx = jnp.full((_LANES,), c, jnp.int32)
    return plsc.load_gather(ref, (idx, cdx))


_TSLAB = 128
_NSLAB = 1000000 // _TSLAB


def _tr_body(tt_hbm, out_hbm, inb, outb, isem2, osem2):
    w = lax.axis_index("s") * _NC + lax.axis_index("c")
    iota = lax.iota(jnp.int32, _LANES)
    par = (iota & 1) * _H
    ids_g = [iota + g * _LANES for g in range(_GRPS)]
    rh_g = [lax.shift_right_logical(ids_g[g], 1) for g in range(_GRPS)]

    nloc = lax.div(_NSLAB - 1 - w, _NW) + 1

    def in_slice(k, slot):
        s = w + k * _NW
        return (tt_hbm.at[:, pl.ds(s * _TSLAB, _TSLAB)], inb.at[slot])

    for j in range(2):
        @pl.when(j < nloc)
        def _():
            src, dst = in_slice(j, j)
            pltpu.async_copy(src, dst, isem2.at[j])

    def slab_body(k, carry):
        s = w + k * _NW
        sl = lax.rem(k, 2)
        src, dst = in_slice(k, sl)
        pltpu.make_async_copy(src, dst, isem2.at[sl]).wait()

        @pl.when(k >= 2)
        def _():
            ps = w + (k - 2) * _NW
            pltpu.make_async_copy(outb.at[sl],
                                  out_hbm.at[pl.ds(ps * (_TSLAB // 2),
                                                   _TSLAB // 2)],
                                  osem2.at[sl]).wait()

        @plsc.parallel_loop(0, _H, unroll=4)
        def tr(h):
            hd = h + iota
            hd = jnp.where(hd >= _H, hd - _H, hd)
            cc = hd + par
            for g in range(_GRPS):
                x = plsc.load_gather(inb.at[sl], (hd, ids_g[g]))
                plsc.store_scatter(outb.at[sl], (rh_g[g], cc), x)

        del tr
        pltpu.async_copy(outb.at[sl],
                         out_hbm.at[pl.ds(s * (_TSLAB // 2), _TSLAB // 2)],
                         osem2.at[sl])

        @pl.when(k + 2 < nloc)
        def _():
            src2, dst2 = in_slice(k + 2, sl)
            pltpu.async_copy(src2, dst2, isem2.at[sl])

        return carry

    lax.fori_loop(0, nloc, slab_body, 0)

    for j in range(2):
        kk = nloc - 2 + j

        @pl.when(kk >= 0)
        def _():
            ps = w + kk * _NW
            pltpu.make_async_copy(
                outb.at[lax.rem(kk, 2)],
                out_hbm.at[pl.ds(ps * (_TSLAB // 2), _TSLAB // 2)],
                osem2.at[lax.rem(kk, 2)]).wait()


@jax.jit
def _tr_call(table_t):
    mesh = plsc.VectorSubcoreMesh(
        core_axis_name="c", subcore_axis_name="s",
        num_cores=_NC, num_subcores=_NS)
    fn = pl.kernel(
        _tr_body,
        out_type=jax.ShapeDtypeStruct((500000, 128), jnp.float32),
        mesh=mesh,
        compiler_params=pltpu.CompilerParams(
            needs_layout_passes=False, use_tc_tiling_on_sc=True),
        scratch_types=[
            pltpu.VMEM((2, _H, _TSLAB), jnp.float32),
            pltpu.VMEM((2, _TSLAB // 2, 128), jnp.float32),
            pltpu.SemaphoreType.DMA((2,)),
            pltpu.SemaphoreType.DMA((2,)),
        ],
    )
    return fn(table_t)


def _sc_body(ids_hbm, table_hbm, pos_hbm, gamma_hbm, beta_hbm, out_hbm,
             pos_v, gb_v, idxr, bufs, tiles, isem, gsem, osem):
    w = lax.axis_index("s") * _NC + lax.axis_index("c")
    m0 = w * _IPW

    pltpu.sync_copy(pos_hbm, pos_v)
    pltpu.sync_copy(gamma_hbm, gb_v.at[0])
    pltpu.sync_copy(beta_hbm, gb_v.at[1])

    iota = lax.iota(jnp.int32, _LANES)
    rows_g = [iota + g * _LANES for g in range(_GRPS)]

    def lbt(m):
        return lax.div(m, _BT), lax.rem(m, _BT)

    def issue_idx(m, slot):
        l, bt = lbt(m)
        pltpu.async_copy(ids_hbm.at[l, pl.ds(bt * 128, 128)],
                         idxr.at[slot], isem.at[slot])

    def issue_gather(slot):
        pltpu.async_copy(table_hbm.at[idxr.at[slot]], bufs.at[slot],
                         gsem.at[slot])

    for j in range(_NBUF):
        issue_idx(m0 + j, j)
    for j in range(3):
        pltpu.make_async_copy(ids_hbm.at[0, pl.ds(0, 128)], idxr.at[j],
                              isem.at[j]).wait()
        issue_gather(j)

    def item_body(i, carry):
        b = lax.rem(i, _NBUF)
        t = lax.rem(i, _NT)
        m = m0 + i
        l, bt = lbt(m)

        pltpu.make_async_copy(table_hbm.at[idxr.at[b]], bufs.at[b],
                              gsem.at[b]).wait()

        @pl.when(i >= _NT)
        def _():
            pltpu.make_async_copy(table_hbm.at[pl.ds(0, 128)], bufs.at[b],
                                  osem.at[t]).wait()

        zero = jnp.zeros((_LANES,), jnp.float32)

        lsplat = jnp.full((_LANES,), l, jnp.int32)

        def half_pass1(g0):
            init = (tuple(zero for _ in range(4)),
                    tuple(zero for _ in range(4)))

            @plsc.parallel_loop(0, _H, unroll=4, carry=init)
            def hp(h, c):
                sums, sqs = c
                hd = h + iota
                hd = jnp.where(hd >= _H, hd - _H, hd)
                p = plsc.load_gather(pos_v, (lsplat, hd))
                ns, nq = [], []
                for k in range(4):
                    g = g0 + k
                    x = plsc.load_gather(bufs.at[b], (rows_g[g], hd)) + p
                    plsc.store_scatter(tiles.at[t], (hd, rows_g[g]), x)
                    ns.append(sums[k] + x)
                    nq.append(sqs[k] + x * x)
                return (tuple(ns), tuple(nq))

            return hp

        s_lo, q_lo = half_pass1(0)
        s_hi, q_hi = half_pass1(4)
        sums = tuple(s_lo) + tuple(s_hi)
        sqs = tuple(q_lo) + tuple(q_hi)

        avs, cvs = [], []
        for g in range(_GRPS):
            mean = sums[g] * (1.0 / _H)
            var = jnp.maximum(sqs[g] * (1.0 / _H) - mean * mean, 0.0)
            a = _rsqrt_vec(var + _EPS)
            avs.append(a)
            cvs.append(mean * a)

        @plsc.parallel_loop(0, _H, unroll=4)
        def pass2(h):
            gam = _splat(gb_v, 0, h)
            bet = _splat(gb_v, 1, h)
            for g in range(_GRPS):
                x = tiles[t, h, pl.ds(g * _LANES, _LANES)]
                y = (x * avs[g] - cvs[g]) * gam + bet
                tiles[t, h, pl.ds(g * _LANES, _LANES)] = y

        del pass2

        for ht in range(8):
            pltpu.async_copy(tiles.at[t, pl.ds(ht * 8, 8)],
                             out_hbm.at[l, ht, bt], osem.at[t])

        @pl.when(i + 3 < _IPW)
        def _():
            nb = lax.rem(i + 3, _NBUF)
            pltpu.make_async_copy(ids_hbm.at[0, pl.ds(0, 128)], idxr.at[nb],
                                  isem.at[nb]).wait()
            issue_gather(nb)

        @pl.when(i + 4 < _IPW)
        def _():
            issue_idx(m0 + i + 4, lax.rem(i + 4, _NBUF))

        return carry

    lax.fori_loop(0, _IPW, item_body, 0)

    for j in range(_NT):
        t = (_IPW - _NT + j) % _NT
        pltpu.make_async_copy(table_hbm.at[pl.ds(0, 128)], bufs.at[0],
                              osem.at[t]).wait()


@jax.jit
def _sc_call(ids_t, item_table, pos_table, ln_gamma, ln_beta):
    mesh = plsc.VectorSubcoreMesh(
        core_axis_name="c", subcore_axis_name="s",
        num_cores=_NC, num_subcores=_NS)
    fn = pl.kernel(
        _sc_body,
        out_type=jax.ShapeDtypeStruct((_L, _H // 8, _BT, 8, 128),
                                      jnp.float32),
        mesh=mesh,
        compiler_params=pltpu.CompilerParams(
            needs_layout_passes=False, use_tc_tiling_on_sc=False),
        scratch_types=[
            pltpu.VMEM((_L, _H), jnp.float32),
            pltpu.VMEM((2, _H), jnp.float32),
            pltpu.VMEM((_NBUF, 128), jnp.int32),
            pltpu.VMEM((_NBUF, 128, _H), jnp.float32),
            pltpu.VMEM((_NT, _H, 128), jnp.float32),
            pltpu.SemaphoreType.DMA((_NBUF,)),
            pltpu.SemaphoreType.DMA((_NBUF,)),
            pltpu.SemaphoreType.DMA((_NT,)),
        ],
    )
    return fn(ids_t, item_table, pos_table, ln_gamma, ln_beta)


def kernel(input_ids, item_table, pos_table, ln_gamma, ln_beta):
    trout = _tr_call(item_table.T)
    tail = item_table[_NSLAB * _TSLAB:].reshape(_TSLAB // 4, 128)
    trout = lax.dynamic_update_slice(trout, tail, (_NSLAB * _TSLAB // 2, 0))
    table_lin = trout.reshape(1000000, _H)
    out5 = _sc_call(input_ids.T, table_lin, pos_table, ln_gamma, ln_beta)
    return out5.transpose(2, 4, 0, 1, 3).reshape(_B, _L, _H)
